# baseline (speedup 1.0000x reference)
import functools

import jax
import jax.numpy as jnp
from jax import lax
from jax.experimental import pallas as pl
from jax.experimental.pallas import tpu as pltpu
from jax.experimental.pallas import tpu_sc as plsc

B = 16
L = 2048
V = 320
D = 1024
LANES = 16
NCHUNK = V // LANES


def _sc_raw_weights(vx, vy, freqs, freqs_t):
  mesh = plsc.VectorSubcoreMesh(core_axis_name="c", subcore_axis_name="s")

  @functools.partial(
      pl.kernel,
      mesh=mesh,
      compiler_params=pltpu.CompilerParams(needs_layout_passes=False),
      out_type=(
          jax.ShapeDtypeStruct((B, L), jnp.float32),
          jax.ShapeDtypeStruct((B, L), jnp.float32),
      ),
      scratch_types=[
          pltpu.VMEM((L,), jnp.int32),
          pltpu.VMEM((L,), jnp.int32),
          pltpu.VMEM((2 * V,), jnp.float32),
          pltpu.VMEM((2 * V,), jnp.float32),
          pltpu.VMEM((L,), jnp.float32),
          pltpu.VMEM((L,), jnp.float32),
          pltpu.VMEM((LANES, V), jnp.float32),
          pltpu.VMEM((LANES,), jnp.float32),
          pltpu.VMEM_SHARED((2 * V,), jnp.float32),
      ],
  )
  def body(vx_h, vy_h, fq_h, fqt_h, wl_h, wg_h, vxv, vyv, cnt, gc_v, wlv,
           wgv, frow, accv, gc_sh):
    c = lax.axis_index("c")
    s = lax.axis_index("s")
    lane = lax.iota(jnp.int32, LANES)
    zero16 = jnp.zeros((LANES,), jnp.float32)
    ones = jnp.ones((LANES,), jnp.float32)

    def allsum(x):
      for k in (1, 2, 4, 8):
        x = x + x.at[lane ^ k].get(mode="promise_in_bounds")
      return x

    def _rowsum_unit(src_h, row_base, dst_base):
      pltpu.sync_copy(src_h.at[pl.ds(row_base, LANES), :], frow)

      def rowbody(r, rs):
        def jb(j, acc):
          return acc + frow[r, pl.ds(LANES * j, LANES)]

        acc = lax.fori_loop(0, NCHUNK, jb, zero16, unroll=4)
        return jnp.where(lane == r, allsum(acc), rs)

      rs = lax.fori_loop(0, LANES, rowbody, zero16)
      accv[...] = rs
      pltpu.sync_copy(accv, gc_sh.at[pl.ds(dst_base, LANES)])

    def _unit(u):
      @pl.when(u < NCHUNK)
      def _row():
        _rowsum_unit(fq_h, LANES * u, LANES * u)

      @pl.when(u >= NCHUNK)
      def _col():
        base = LANES * (u - NCHUNK)
        _rowsum_unit(fqt_h, base, V + base)

    @pl.when(s < 8)
    def _hist():
      b = c * 8 + s
      pltpu.sync_copy(vx_h.at[b], vxv)
      pltpu.sync_copy(vy_h.at[b], vyv)

      def zb(j, _):
        cnt[pl.ds(LANES * j, LANES)] = zero16
        return 0

      lax.fori_loop(0, 2 * V // LANES, zb, 0, unroll=4)

      def sb(i, _):
        ix = vxv[pl.ds(LANES * i, LANES)]
        iy = vyv[pl.ds(LANES * i, LANES)]
        plsc.addupdate_scatter(cnt, [ix], ones)
        plsc.addupdate_scatter(cnt, [iy + V], ones)
        return 0

      lax.fori_loop(0, L // LANES, sb, 0, unroll=4)

      _unit(2 * s)
      _unit(2 * s + 1)

    @pl.when(s >= 8)
    def _freq_work():
      for k in range(3):
        _unit(2 * 8 + (s - 8) * 3 + k)

    plsc.subcore_barrier()

    @pl.when(s < 8)
    def _weights():
      b = c * 8 + s
      pltpu.sync_copy(gc_sh, gc_v)

      def gb(i, _):
        ix = vxv[pl.ds(LANES * i, LANES)]
        iy = vyv[pl.ds(LANES * i, LANES)] + V
        fx = plsc.load_gather(cnt, [ix])
        fy = plsc.load_gather(cnt, [iy])
        gx = plsc.load_gather(gc_v, [ix])
        gy = plsc.load_gather(gc_v, [iy])
        wlv[pl.ds(LANES * i, LANES)] = 1.0 / (fx + fy)
        wgv[pl.ds(LANES * i, LANES)] = 1.0 / (gx + gy)
        return 0

      lax.fori_loop(0, L // LANES, gb, 0, unroll=4)

      pltpu.sync_copy(wlv, wl_h.at[b])
      pltpu.sync_copy(wgv, wg_h.at[b])

  return body(vx, vy, freqs, freqs_t)


def _tc_pool(feat4, wl, wg):
  def body(f_ref, wl_ref, wg_ref, o_ref):
    wlr = wl_ref[0]
    wgr = wg_ref[0]
    p = wlr * wgr * (1.0 / (jnp.sum(wlr) * jnp.sum(wgr)))
    e = jnp.exp(p)
    a = e * (1.0 / jnp.sum(e))
    o_ref[...] = jnp.dot(a, f_ref[0, 0],
                         preferred_element_type=jnp.float32)[None]

  out3 = pl.pallas_call(
      body,
      grid=(B,),
      in_specs=[
          pl.BlockSpec((1, 1, L, D), lambda b: (b, 1, 0, 0)),
          pl.BlockSpec((1, 1, L), lambda b: (b, 0, 0)),
          pl.BlockSpec((1, 1, L), lambda b: (b, 0, 0)),
      ],
      out_specs=pl.BlockSpec((1, 1, D), lambda b: (b, 0, 0)),
      out_shape=jax.ShapeDtypeStruct((B, 1, D), jnp.float32),
  )(feat4, wl.reshape(B, 1, L), wg.reshape(B, 1, L))
  return out3.reshape(B, D)


def kernel(input_feature, input_lengths, vq_indices, freqs):
  del input_lengths
  vx = vq_indices[:, :, 0]
  vy = vq_indices[:, :, 1]
  wl, wg = _sc_raw_weights(vx, vy, freqs, freqs.T)
  return _tc_pool(input_feature, wl, wg)


# speedup vs baseline: 1.0441x; 1.0441x over previous
import functools

import jax
import jax.numpy as jnp
from jax import lax
from jax.experimental import pallas as pl
from jax.experimental.pallas import tpu as pltpu
from jax.experimental.pallas import tpu_sc as plsc

B = 16
L = 2048
V = 320
D = 1024
LANES = 16


def _tc_freq_sums(freqs):
  def body(f_ref, o_ref):
    f = f_ref[...]
    o_ref[...] = jnp.stack([jnp.sum(f, axis=1), jnp.sum(f, axis=0)])

  return pl.pallas_call(
      body,
      out_shape=jax.ShapeDtypeStruct((2, V), jnp.float32),
  )(freqs)


def _sc_raw_weights(vx, vy, gsums):
  mesh = plsc.VectorSubcoreMesh(core_axis_name="c", subcore_axis_name="s")

  @functools.partial(
      pl.kernel,
      mesh=mesh,
      compiler_params=pltpu.CompilerParams(needs_layout_passes=False),
      out_type=(
          jax.ShapeDtypeStruct((B, L), jnp.float32),
          jax.ShapeDtypeStruct((B, L), jnp.float32),
      ),
      scratch_types=[
          pltpu.VMEM((L,), jnp.int32),
          pltpu.VMEM((L,), jnp.int32),
          pltpu.VMEM((2 * V,), jnp.float32),
          pltpu.VMEM((2 * V,), jnp.float32),
          pltpu.VMEM((L,), jnp.float32),
          pltpu.VMEM((L,), jnp.float32),
          pltpu.SemaphoreType.DMA,
          pltpu.SemaphoreType.DMA,
          pltpu.SemaphoreType.DMA,
          pltpu.SemaphoreType.DMA,
      ],
  )
  def body(vx_h, vy_h, gs_h, wl_h, wg_h, vxv, vyv, cnt, gc_v, wlv, wgv,
           sem1, sem2, sem3, sem4):
    c = lax.axis_index("c")
    s = lax.axis_index("s")
    zero16 = jnp.zeros((LANES,), jnp.float32)
    ones = jnp.ones((LANES,), jnp.float32)

    @pl.when(s < 8)
    def _work():
      b = c * 8 + s
      cp1 = pltpu.async_copy(vx_h.at[b], vxv, sem1)
      cp2 = pltpu.async_copy(vy_h.at[b], vyv, sem2)
      cp3 = pltpu.async_copy(gs_h, gc_v, sem3)

      def zb(j, _):
        cnt[pl.ds(LANES * j, LANES)] = zero16
        return 0

      lax.fori_loop(0, 2 * V // LANES, zb, 0, unroll=4)
      cp1.wait()
      cp2.wait()

      def sb(i, _):
        ix = vxv[pl.ds(LANES * i, LANES)]
        iy = vyv[pl.ds(LANES * i, LANES)]
        plsc.addupdate_scatter(cnt, [ix], ones)
        plsc.addupdate_scatter(cnt, [iy + V], ones)
        return 0

      lax.fori_loop(0, L // LANES, sb, 0, unroll=4)
      cp3.wait()

      def gb(i, _):
        ix = vxv[pl.ds(LANES * i, LANES)]
        iy = vyv[pl.ds(LANES * i, LANES)] + V
        fx = plsc.load_gather(cnt, [ix])
        fy = plsc.load_gather(cnt, [iy])
        gx = plsc.load_gather(gc_v, [ix])
        gy = plsc.load_gather(gc_v, [iy])
        wlv[pl.ds(LANES * i, LANES)] = 1.0 / (fx + fy)
        wgv[pl.ds(LANES * i, LANES)] = 1.0 / (gx + gy)
        return 0

      lax.fori_loop(0, L // LANES, gb, 0, unroll=4)

      cp4 = pltpu.async_copy(wlv, wl_h.at[b], sem4)
      pltpu.sync_copy(wgv, wg_h.at[b])
      cp4.wait()

  return body(vx, vy, gsums)


def _tc_pool(feat4, wl, wg):
  def body(f_ref, wl_ref, wg_ref, o_ref):
    wlr = wl_ref[0]
    wgr = wg_ref[0]
    p = wlr * wgr * (1.0 / (jnp.sum(wlr) * jnp.sum(wgr)))
    e = jnp.exp(p)
    a = e * (1.0 / jnp.sum(e))
    o_ref[...] = jnp.dot(a, f_ref[0, 0],
                         preferred_element_type=jnp.float32)[None]

  out3 = pl.pallas_call(
      body,
      grid=(B,),
      in_specs=[
          pl.BlockSpec((1, 1, L, D), lambda b: (b, 1, 0, 0)),
          pl.BlockSpec((1, 1, L), lambda b: (b, 0, 0)),
          pl.BlockSpec((1, 1, L), lambda b: (b, 0, 0)),
      ],
      out_specs=pl.BlockSpec((1, 1, D), lambda b: (b, 0, 0)),
      out_shape=jax.ShapeDtypeStruct((B, 1, D), jnp.float32),
  )(feat4, wl.reshape(B, 1, L), wg.reshape(B, 1, L))
  return out3.reshape(B, D)


def kernel(input_feature, input_lengths, vq_indices, freqs):
  del input_lengths
  vx = vq_indices[:, :, 0]
  vy = vq_indices[:, :, 1]
  gsums = _tc_freq_sums(freqs).reshape(2 * V)
  wl, wg = _sc_raw_weights(vx, vy, gsums)
  return _tc_pool(input_feature, wl, wg)


# speedup vs baseline: 1.0589x; 1.0142x over previous
import functools

import jax
import jax.numpy as jnp
from jax import lax
from jax.experimental import pallas as pl
from jax.experimental.pallas import tpu as pltpu
from jax.experimental.pallas import tpu_sc as plsc

B = 16
L = 2048
V = 320
D = 1024
LANES = 16


def _tc_freq_sums(freqs):
  def body(f_ref, o_ref):
    f = f_ref[...]
    o_ref[...] = jnp.stack([jnp.sum(f, axis=1), jnp.sum(f, axis=0)])

  return pl.pallas_call(
      body,
      out_shape=jax.ShapeDtypeStruct((2, V), jnp.float32),
  )(freqs)


def _sc_raw_weights(vx, vy, gsums):
  mesh = plsc.VectorSubcoreMesh(core_axis_name="c", subcore_axis_name="s")

  @functools.partial(
      pl.kernel,
      mesh=mesh,
      compiler_params=pltpu.CompilerParams(needs_layout_passes=False),
      out_type=(
          jax.ShapeDtypeStruct((B, L), jnp.float32),
          jax.ShapeDtypeStruct((B, L), jnp.float32),
      ),
      scratch_types=[
          pltpu.VMEM((L // 2,), jnp.int32),
          pltpu.VMEM((L // 2,), jnp.int32),
          pltpu.VMEM((2 * V,), jnp.float32),
          pltpu.VMEM((2 * V,), jnp.float32),
          pltpu.VMEM((2 * V,), jnp.float32),
          pltpu.VMEM((L // 2,), jnp.float32),
          pltpu.VMEM((L // 2,), jnp.float32),
          pltpu.VMEM_SHARED((16, 2 * V), jnp.float32),
          pltpu.SemaphoreType.DMA,
          pltpu.SemaphoreType.DMA,
          pltpu.SemaphoreType.DMA,
          pltpu.SemaphoreType.DMA,
      ],
  )
  def body(vx_h, vy_h, gs_h, wl_h, wg_h, vxv, vyv, cnt, pcnt, gc_v, wlv,
           wgv, xch, sem1, sem2, sem3, sem4):
    c = lax.axis_index("c")
    s = lax.axis_index("s")
    zero16 = jnp.zeros((LANES,), jnp.float32)
    ones = jnp.ones((LANES,), jnp.float32)
    H = L // 2

    b = c * 8 + lax.rem(s, 8)
    half = s // 8
    off = H * half
    partner = lax.rem(s + 8, 16)

    cp1 = pltpu.async_copy(vx_h.at[b, pl.ds(off, H)], vxv, sem1)
    cp2 = pltpu.async_copy(vy_h.at[b, pl.ds(off, H)], vyv, sem2)
    cp3 = pltpu.async_copy(gs_h, gc_v, sem3)

    def zb(j, _):
      cnt[pl.ds(LANES * j, LANES)] = zero16
      return 0

    lax.fori_loop(0, 2 * V // LANES, zb, 0, unroll=4)
    cp1.wait()
    cp2.wait()

    def sb(i, _):
      ix = vxv[pl.ds(LANES * i, LANES)]
      iy = vyv[pl.ds(LANES * i, LANES)]
      plsc.addupdate_scatter(cnt, [ix], ones)
      plsc.addupdate_scatter(cnt, [iy + V], ones)
      return 0

    lax.fori_loop(0, H // LANES, sb, 0, unroll=4)

    pltpu.sync_copy(cnt, xch.at[s])
    plsc.subcore_barrier()
    pltpu.sync_copy(xch.at[partner], pcnt)

    def mb(j, _):
      sl = pl.ds(LANES * j, LANES)
      cnt[sl] = cnt[sl] + pcnt[sl]
      return 0

    lax.fori_loop(0, 2 * V // LANES, mb, 0, unroll=4)
    cp3.wait()

    def gb(i, _):
      ix = vxv[pl.ds(LANES * i, LANES)]
      iy = vyv[pl.ds(LANES * i, LANES)] + V
      fx = plsc.load_gather(cnt, [ix])
      fy = plsc.load_gather(cnt, [iy])
      gx = plsc.load_gather(gc_v, [ix])
      gy = plsc.load_gather(gc_v, [iy])
      wlv[pl.ds(LANES * i, LANES)] = 1.0 / (fx + fy)
      wgv[pl.ds(LANES * i, LANES)] = 1.0 / (gx + gy)
      return 0

    lax.fori_loop(0, H // LANES, gb, 0, unroll=4)

    cp4 = pltpu.async_copy(wlv, wl_h.at[b, pl.ds(off, H)], sem4)
    pltpu.sync_copy(wgv, wg_h.at[b, pl.ds(off, H)])
    cp4.wait()

  return body(vx, vy, gsums)


def _tc_pool(feat4, wl, wg):
  def body(f_ref, wl_ref, wg_ref, o_ref):
    wlr = wl_ref[0]
    wgr = wg_ref[0]
    p = wlr * wgr * (1.0 / (jnp.sum(wlr) * jnp.sum(wgr)))
    e = jnp.exp(p)
    a = e * (1.0 / jnp.sum(e))
    o_ref[...] = jnp.dot(a, f_ref[0, 0],
                         preferred_element_type=jnp.float32)[None]

  out3 = pl.pallas_call(
      body,
      grid=(B,),
      in_specs=[
          pl.BlockSpec((1, 1, L, D), lambda b: (b, 1, 0, 0)),
          pl.BlockSpec((1, 1, L), lambda b: (b, 0, 0)),
          pl.BlockSpec((1, 1, L), lambda b: (b, 0, 0)),
      ],
      out_specs=pl.BlockSpec((1, 1, D), lambda b: (b, 0, 0)),
      out_shape=jax.ShapeDtypeStruct((B, 1, D), jnp.float32),
  )(feat4, wl.reshape(B, 1, L), wg.reshape(B, 1, L))
  return out3.reshape(B, D)


def kernel(input_feature, input_lengths, vq_indices, freqs):
  del input_lengths
  vx = vq_indices[:, :, 0]
  vy = vq_indices[:, :, 1]
  gsums = _tc_freq_sums(freqs).reshape(2 * V)
  wl, wg = _sc_raw_weights(vx, vy, gsums)
  return _tc_pool(input_feature, wl, wg)


# speedup vs baseline: 1.1061x; 1.0446x over previous
import functools

import jax
import jax.numpy as jnp
from jax import lax
from jax.experimental import pallas as pl
from jax.experimental.pallas import tpu as pltpu
from jax.experimental.pallas import tpu_sc as plsc

B = 16
L = 2048
V = 320
D = 1024
LANES = 16


def _tc_freq_sums(freqs):
  def body(f_ref, o_ref):
    f = f_ref[...]
    o_ref[...] = jnp.stack([jnp.sum(f, axis=1), jnp.sum(f, axis=0)])

  return pl.pallas_call(
      body,
      out_shape=jax.ShapeDtypeStruct((2, V), jnp.float32),
  )(freqs)


def _sc_local_weights(vx, vy):
  mesh = plsc.VectorSubcoreMesh(core_axis_name="c", subcore_axis_name="s")

  @functools.partial(
      pl.kernel,
      mesh=mesh,
      compiler_params=pltpu.CompilerParams(needs_layout_passes=False),
      out_type=jax.ShapeDtypeStruct((B, L), jnp.float32),
      scratch_types=[
          pltpu.VMEM((L // 2,), jnp.int32),
          pltpu.VMEM((L // 2,), jnp.int32),
          pltpu.VMEM((2 * V,), jnp.float32),
          pltpu.VMEM((2 * V,), jnp.float32),
          pltpu.VMEM((L // 2,), jnp.float32),
          pltpu.VMEM_SHARED((16, 2 * V), jnp.float32),
          pltpu.SemaphoreType.DMA,
          pltpu.SemaphoreType.DMA,
      ],
  )
  def body(vx_h, vy_h, wl_h, vxv, vyv, cnt, pcnt, wlv, xch, sem1, sem2):
    c = lax.axis_index("c")
    s = lax.axis_index("s")
    zero16 = jnp.zeros((LANES,), jnp.float32)
    ones = jnp.ones((LANES,), jnp.float32)
    H = L // 2

    b = c * 8 + lax.rem(s, 8)
    off = H * (s // 8)
    partner = lax.rem(s + 8, 16)

    cp1 = pltpu.async_copy(vx_h.at[b, pl.ds(off, H)], vxv, sem1)
    cp2 = pltpu.async_copy(vy_h.at[b, pl.ds(off, H)], vyv, sem2)

    def zb(j, _):
      cnt[pl.ds(LANES * j, LANES)] = zero16
      return 0

    lax.fori_loop(0, 2 * V // LANES, zb, 0, unroll=4)
    cp1.wait()
    cp2.wait()

    def sb(i, _):
      ix = vxv[pl.ds(LANES * i, LANES)]
      iy = vyv[pl.ds(LANES * i, LANES)]
      plsc.addupdate_scatter(cnt, [ix], ones)
      plsc.addupdate_scatter(cnt, [iy + V], ones)
      return 0

    lax.fori_loop(0, H // LANES, sb, 0, unroll=4)

    pltpu.sync_copy(cnt, xch.at[s])
    plsc.subcore_barrier()
    pltpu.sync_copy(xch.at[partner], pcnt)

    def mb(j, _):
      sl = pl.ds(LANES * j, LANES)
      cnt[sl] = cnt[sl] + pcnt[sl]
      return 0

    lax.fori_loop(0, 2 * V // LANES, mb, 0, unroll=4)

    def gb(i, _):
      ix = vxv[pl.ds(LANES * i, LANES)]
      iy = vyv[pl.ds(LANES * i, LANES)] + V
      fx = plsc.load_gather(cnt, [ix])
      fy = plsc.load_gather(cnt, [iy])
      wlv[pl.ds(LANES * i, LANES)] = 1.0 / (fx + fy)
      return 0

    lax.fori_loop(0, H // LANES, gb, 0, unroll=4)

    pltpu.sync_copy(wlv, wl_h.at[b, pl.ds(off, H)])

  return body(vx, vy)


def _tc_pool(feat4, wl, gsums2, vx, vy):
  def body(f_ref, wl_ref, g_ref, vx_ref, vy_ref, o_ref):
    wlr = wl_ref[0]
    vxr = vx_ref[0]
    vyr = vy_ref[0]
    riota = lax.broadcasted_iota(jnp.int32, (V, L), 0)
    eqx = jnp.where(riota == vxr, 1.0, 0.0)
    eqy = jnp.where(riota == vyr, 1.0, 0.0)
    gx = jnp.dot(g_ref[0:1, :], eqx, preferred_element_type=jnp.float32)
    gy = jnp.dot(g_ref[1:2, :], eqy, preferred_element_type=jnp.float32)
    wgr = 1.0 / (gx + gy)
    p = wlr * wgr * (1.0 / (jnp.sum(wlr) * jnp.sum(wgr)))
    e = jnp.exp(p)
    a = e * (1.0 / jnp.sum(e))
    o_ref[...] = jnp.dot(a, f_ref[0, 0],
                         preferred_element_type=jnp.float32)[None]

  out3 = pl.pallas_call(
      body,
      grid=(B,),
      in_specs=[
          pl.BlockSpec((1, 1, L, D), lambda b: (b, 1, 0, 0)),
          pl.BlockSpec((1, 1, L), lambda b: (b, 0, 0)),
          pl.BlockSpec((2, V), lambda b: (0, 0)),
          pl.BlockSpec((1, 1, L), lambda b: (b, 0, 0)),
          pl.BlockSpec((1, 1, L), lambda b: (b, 0, 0)),
      ],
      out_specs=pl.BlockSpec((1, 1, D), lambda b: (b, 0, 0)),
      out_shape=jax.ShapeDtypeStruct((B, 1, D), jnp.float32),
  )(feat4, wl.reshape(B, 1, L), gsums2,
    vx.reshape(B, 1, L), vy.reshape(B, 1, L))
  return out3.reshape(B, D)


def kernel(input_feature, input_lengths, vq_indices, freqs):
  del input_lengths
  vx = vq_indices[:, :, 0]
  vy = vq_indices[:, :, 1]
  gsums2 = _tc_freq_sums(freqs)
  wl = _sc_local_weights(vx, vy)
  return _tc_pool(input_feature, wl, gsums2, vx, vy)
